# baseline (speedup 1.0000x reference)
import jax
import jax.numpy as jnp
from jax import lax
from jax.experimental import pallas as pl
from jax.experimental.pallas import tpu as pltpu
from jax.experimental.pallas import tpu_sc as plsc

_BATCH = 16384
_D = 32
_NC = 2
_NS = 16
_NW = _NC * _NS
_BPW = _BATCH // _NW
_CHUNK = 128
_NCHUNK = _BPW // _CHUNK
_L = 16


def _body(uidx_hbm, iidx_hbm, utab_hbm, itab_hbm, out_hbm,
          uidx_v, iidx_v, urows_v, irows_v, out_v, sem):
    c = lax.axis_index("c")
    s = lax.axis_index("s")
    wid = s * _NC + c

    pltpu.sync_copy(uidx_hbm.at[wid], uidx_v)
    pltpu.sync_copy(iidx_hbm.at[wid], iidx_v)

    copies = []
    for j in range(_NCHUNK):
        copies.append(pltpu.async_copy(
            utab_hbm.at[uidx_v.at[j]],
            urows_v.at[pl.ds(j * _CHUNK, _CHUNK)], sem))
        copies.append(pltpu.async_copy(
            itab_hbm.at[iidx_v.at[j]],
            irows_v.at[pl.ds(j * _CHUNK, _CHUNK)], sem))
    for cp in copies:
        cp.wait()

    def blk_body(blk, carry):
        rows = blk * _L + lax.iota(jnp.int32, _L)
        acc = jnp.zeros((_L,), jnp.float32)
        for d in range(_D):
            col = jnp.full((_L,), d, jnp.int32)
            u = plsc.load_gather(urows_v, [rows, col])
            v = plsc.load_gather(irows_v, [rows, col])
            acc = acc + u * v
        out_v[pl.ds(blk * _L, _L)] = acc
        return carry

    lax.fori_loop(0, _BPW // _L, blk_body, 0)
    pltpu.sync_copy(out_v, out_hbm.at[pl.ds(wid * _BPW, _BPW)])


@jax.jit
def kernel(user_indices, item_indices, user_table, item_table):
    uidx = user_indices.astype(jnp.int32).reshape(_NW, _NCHUNK, _CHUNK)
    iidx = item_indices.astype(jnp.int32).reshape(_NW, _NCHUNK, _CHUNK)
    mesh = plsc.VectorSubcoreMesh(core_axis_name="c", subcore_axis_name="s")
    f = pl.kernel(
        _body,
        out_type=jax.ShapeDtypeStruct((_BATCH,), jnp.float32),
        mesh=mesh,
        compiler_params=pltpu.CompilerParams(
            needs_layout_passes=False, use_tc_tiling_on_sc=False),
        scratch_types=[
            pltpu.VMEM((_NCHUNK, _CHUNK), jnp.int32),
            pltpu.VMEM((_NCHUNK, _CHUNK), jnp.int32),
            pltpu.VMEM((_BPW, _D), jnp.float32),
            pltpu.VMEM((_BPW, _D), jnp.float32),
            pltpu.VMEM((_BPW,), jnp.float32),
            pltpu.SemaphoreType.DMA,
        ],
    )
    return f(uidx, iidx, user_table, item_table)
